# speedup vs baseline: 19.1453x; 19.1453x over previous
import jax
import jax.numpy as jnp
from jax.experimental import pallas as pl

TILES = 6


def _sage_body(x_ref, rdeg_ref, ws_ref, wn_ref, b_ref, o_ref):
    x = x_ref[0]
    H, W, F = x.shape
    zrow = jnp.zeros((1, W, F), x.dtype)
    zcol = jnp.zeros((H, 1, F), x.dtype)
    agg = (
        jnp.concatenate([x[1:], zrow], axis=0)
        + jnp.concatenate([zrow, x[:-1]], axis=0)
        + jnp.concatenate([x[:, 1:], zcol], axis=1)
        + jnp.concatenate([zcol, x[:, :-1]], axis=1)
    )
    agg = agg * rdeg_ref[0][:, :, None]
    xm = x.reshape(H * W, F)
    am = agg.reshape(H * W, F)
    out = jnp.dot(xm, ws_ref[...], preferred_element_type=jnp.float32)
    out = out + jnp.dot(am, wn_ref[...], preferred_element_type=jnp.float32)
    out = out + b_ref[...]
    out = jnp.maximum(out, 0.0)
    o_ref[0] = out.reshape(H, W, -1)


def _sage(x, rdeg, Ws, Wn, b):
    T, H, W, F = x.shape
    Fo = Ws.shape[-1]
    return pl.pallas_call(
        _sage_body,
        grid=(T,),
        in_specs=[
            pl.BlockSpec((1, H, W, F), lambda t: (t, 0, 0, 0)),
            pl.BlockSpec((1, H, W), lambda t: (t, 0, 0)),
            pl.BlockSpec((F, Fo), lambda t: (0, 0)),
            pl.BlockSpec((F, Fo), lambda t: (0, 0)),
            pl.BlockSpec((1, Fo), lambda t: (0, 0)),
        ],
        out_specs=pl.BlockSpec((1, H, W, Fo), lambda t: (t, 0, 0, 0)),
        out_shape=jax.ShapeDtypeStruct((T, H, W, Fo), jnp.float32),
    )(x, rdeg, Ws, Wn, b)


def _pool_body(x_ref, o_ref):
    x = x_ref[0]
    H, W, F = x.shape
    x = x.reshape(H // 2, 2, W // 2, 2, F)
    o_ref[0] = x.mean(axis=(1, 3))


def _pool(x):
    T, H, W, F = x.shape
    return pl.pallas_call(
        _pool_body,
        grid=(T,),
        in_specs=[pl.BlockSpec((1, H, W, F), lambda t: (t, 0, 0, 0))],
        out_specs=pl.BlockSpec((1, H // 2, W // 2, F), lambda t: (t, 0, 0, 0)),
        out_shape=jax.ShapeDtypeStruct((T, H // 2, W // 2, F), jnp.float32),
    )(x)


def _up_body(x_ref, w_ref, b_ref, o_ref):
    x = x_ref[0]
    h, w, F = x.shape
    y = jnp.dot(x.reshape(h * w, F), w_ref[...], preferred_element_type=jnp.float32)
    y = y + b_ref[...]
    o_ref[0] = y.reshape(h, w, -1)


def _up_matmul(x, Wu, bu):
    T, h, w, F = x.shape
    Fo = Wu.shape[-1]
    return pl.pallas_call(
        _up_body,
        grid=(T,),
        in_specs=[
            pl.BlockSpec((1, h, w, F), lambda t: (t, 0, 0, 0)),
            pl.BlockSpec((F, Fo), lambda t: (0, 0)),
            pl.BlockSpec((1, Fo), lambda t: (0, 0)),
        ],
        out_specs=pl.BlockSpec((1, h, w, Fo), lambda t: (t, 0, 0, 0)),
        out_shape=jax.ShapeDtypeStruct((T, h, w, Fo), jnp.float32),
    )(x, Wu, bu)


def _repeat2(x):
    T, h, w, F = x.shape
    x = jnp.broadcast_to(x[:, :, None, :, None, :], (T, h, 2, w, 2, F))
    return x.reshape(T, 2 * h, 2 * w, F)


def _double_conv(x, rdeg, p):
    s0, n0, b0 = p["c0"]
    s1, n1, b1 = p["c1"]
    x = _sage(x, rdeg, s0, n0, b0.reshape(1, -1))
    return _sage(x, rdeg, s1, n1, b1.reshape(1, -1))


def _unet(x, level, params, rdegs):
    p = params["level%d" % level]
    before = _double_conv(x, rdegs[level], p["conv1"])
    down = _pool(before)
    if level == 1:
        lower = _double_conv(down, rdegs[level + 1], p["lower"])
    else:
        lower = _unet(down, level + 1, params, rdegs)
    up = _repeat2(_up_matmul(lower, p["upW"], p["upb"].reshape(1, -1)))
    x = jnp.concatenate([before, up], axis=-1)
    return _double_conv(x, rdegs[level], p["conv2"])


def kernel(inputs, params, graphs):
    B, T, H, W, F = inputs.shape
    x = inputs.reshape(T, H, W, F)
    rdegs = []
    nx = H
    for g in graphs:
        deg = g[2]
        rdegs.append((1.0 / deg).reshape(T, nx, nx))
        nx //= 2
    out = _unet(x, 0, params, rdegs)
    return out[None]


# speedup vs baseline: 33.8649x; 1.7688x over previous
import jax
import jax.numpy as jnp
from jax.experimental import pallas as pl

TILES = 6


def _nagg(x, rdeg):
    H, W, F = x.shape
    zr = jnp.zeros((1, W, F), x.dtype)
    zc = jnp.zeros((H, 1, F), x.dtype)
    s = (
        jnp.concatenate([x[1:], zr], axis=0)
        + jnp.concatenate([zr, x[:-1]], axis=0)
        + jnp.concatenate([x[:, 1:], zc], axis=1)
        + jnp.concatenate([zc, x[:, :-1]], axis=1)
    )
    return s * rdeg[:, :, None]


def _sage(x, rdeg, Ws, Wn, b):
    H, W, F = x.shape
    a = _nagg(x, rdeg)
    o = jnp.dot(x.reshape(H * W, F), Ws, preferred_element_type=jnp.float32)
    o = o + jnp.dot(a.reshape(H * W, F), Wn, preferred_element_type=jnp.float32)
    o = jnp.maximum(o + b, 0.0)
    return o.reshape(H, W, -1)


def _pool(x):
    H, W, F = x.shape
    return x.reshape(H // 2, 2, W // 2, 2, F).mean(axis=(1, 3))


def _rep2(x):
    h, w, F = x.shape
    x = jnp.broadcast_to(x[:, None, :, None, :], (h, 2, w, 2, F))
    return x.reshape(2 * h, 2 * w, F)


def _unet_body(x_ref, rd0_ref, rd1_ref, rd2_ref, *rest):
    (
        s10, n10, c10, s11, n11, c11,
        s20, n20, c20, s21, n21, c21,
        sl0, nl0, cl0, sl1, nl1, cl1,
        u1w, u1b,
        t20, m20, d20, t21, m21, d21,
        u0w, u0b,
        t10, m10, d10, t11, m11, d11,
        out_ref,
    ) = rest
    x0 = x_ref[0]
    rd0, rd1, rd2 = rd0_ref[0], rd1_ref[0], rd2_ref[0]

    b0 = _sage(_sage(x0, rd0, s10[...], n10[...], c10[...]),
               rd0, s11[...], n11[...], c11[...])
    p0 = _pool(b0)
    b1 = _sage(_sage(p0, rd1, s20[...], n20[...], c20[...]),
               rd1, s21[...], n21[...], c21[...])
    p1 = _pool(b1)
    lo = _sage(_sage(p1, rd2, sl0[...], nl0[...], cl0[...]),
               rd2, sl1[...], nl1[...], cl1[...])

    h2, w2, fl = lo.shape
    up1 = jnp.dot(lo.reshape(h2 * w2, fl), u1w[...],
                  preferred_element_type=jnp.float32) + u1b[...]
    up1 = _rep2(up1.reshape(h2, w2, -1))
    c1 = jnp.concatenate([b1, up1], axis=-1)
    o1 = _sage(_sage(c1, rd1, t20[...], m20[...], d20[...]),
               rd1, t21[...], m21[...], d21[...])

    h1, w1, f1 = o1.shape
    up0 = jnp.dot(o1.reshape(h1 * w1, f1), u0w[...],
                  preferred_element_type=jnp.float32) + u0b[...]
    up0 = _rep2(up0.reshape(h1, w1, -1))
    c0 = jnp.concatenate([b0, up0], axis=-1)
    out_ref[0] = _sage(_sage(c0, rd0, t10[...], m10[...], d10[...]),
                       rd0, t11[...], m11[...], d11[...])


def kernel(inputs, params, graphs):
    B, T, H, W, F = inputs.shape
    x = inputs.reshape(T, H, W, F)
    rdegs = []
    nx = H
    for g in graphs:
        rdegs.append((1.0 / g[2]).reshape(T, nx, nx))
        nx //= 2

    p0, p1 = params["level0"], params["level1"]

    def triple(t):
        Ws, Wn, b = t
        return [Ws, Wn, b.reshape(1, -1)]

    weights = (
        triple(p0["conv1"]["c0"]) + triple(p0["conv1"]["c1"])
        + triple(p1["conv1"]["c0"]) + triple(p1["conv1"]["c1"])
        + triple(p1["lower"]["c0"]) + triple(p1["lower"]["c1"])
        + [p1["upW"], p1["upb"].reshape(1, -1)]
        + triple(p1["conv2"]["c0"]) + triple(p1["conv2"]["c1"])
        + [p0["upW"], p0["upb"].reshape(1, -1)]
        + triple(p0["conv2"]["c0"]) + triple(p0["conv2"]["c1"])
    )

    def tile_spec(a):
        s = a.shape
        return pl.BlockSpec((1,) + s[1:], lambda t: (t,) + (0,) * (len(s) - 1))

    def full_spec(a):
        nd = a.ndim
        return pl.BlockSpec(a.shape, lambda t, _n=nd: (0,) * _n)

    Fo = p0["conv2"]["c1"][0].shape[-1]
    out = pl.pallas_call(
        _unet_body,
        grid=(T,),
        in_specs=[tile_spec(x)] + [tile_spec(r) for r in rdegs]
        + [full_spec(wa) for wa in weights],
        out_specs=pl.BlockSpec((1, H, W, Fo), lambda t: (t, 0, 0, 0)),
        out_shape=jax.ShapeDtypeStruct((T, H, W, Fo), jnp.float32),
    )(x, *rdegs, *weights)
    return out[None]
